# Initial kernel scaffold; baseline (speedup 1.0000x reference)
import jax
import jax.numpy as jnp
from jax.experimental import pallas as pl


def kernel(x, edge_index, edge_weight, W1, b1, W2, b2):
    raise NotImplementedError("write your pallas kernel here")


# speedup vs baseline: 14.0515x; 14.0515x over previous
import functools

import jax
import jax.numpy as jnp
from jax import lax
from jax.experimental import pallas as pl
from jax.experimental.pallas import tpu as pltpu
from jax.experimental.pallas import tpu_sc as plsc

_N = 10000
_D = 128
_E = 320000
_NW = 32
_EPT = _E // _NW
_CH = 80
_NCH = _EPT // _CH
_NPAD = 10240
_ROWS_PER_TILE = _NPAD // 16
_BCH = 25
_NBLK = _NCH // _BCH


def _sc_mesh():
    return plsc.VectorSubcoreMesh(core_axis_name="c", subcore_axis_name="s")


def _deg_body(col_hbm, ew_hbm, deg_hbm, col_v, ew_v, vals, accum):
    c = lax.axis_index("c")
    s = lax.axis_index("s")
    wid = s * 2 + c
    zero = jnp.zeros((16,), jnp.float32)
    for e in range(_CH):
        vals[e, :] = zero
    for k in range(_ROWS_PER_TILE // _CH):
        pltpu.sync_copy(vals, accum.at[pl.ds(s * _ROWS_PER_TILE + k * _CH, _CH)])
    plsc.subcore_barrier()

    def blk(b, carry):
        pltpu.sync_copy(col_hbm.at[wid * _NBLK + b], col_v)
        pltpu.sync_copy(ew_hbm.at[wid * _NBLK + b], ew_v)

        def chunk(j, carry2):
            for g in range(_CH // 16):
                ewv = ew_v[j, pl.ds(g * 16, 16)]
                for l in range(16):
                    vals[g * 16 + l, :] = jnp.full((16,), ewv[l], jnp.float32)
            pltpu.sync_copy(vals, accum.at[col_v.at[j]], add=True)
            return carry2

        lax.fori_loop(0, _BCH, chunk, 0)
        return carry

    lax.fori_loop(0, _NBLK, blk, 0)
    plsc.subcore_barrier()
    pltpu.sync_copy(accum.at[pl.ds(s * _ROWS_PER_TILE, _ROWS_PER_TILE)],
                    deg_hbm.at[c, pl.ds(s * _ROWS_PER_TILE, _ROWS_PER_TILE)])


def _sc_degree(col3, ew3):
    k = pl.kernel(
        _deg_body,
        out_type=jax.ShapeDtypeStruct((2, _NPAD, 16), jnp.float32),
        mesh=_sc_mesh(),
        scratch_types=[
            pltpu.VMEM((_BCH, _CH), jnp.int32),
            pltpu.VMEM((_BCH, _CH), jnp.float32),
            pltpu.VMEM((_CH, 16), jnp.float32),
            pltpu.VMEM_SHARED((_NPAD, 16), jnp.float32),
        ],
    )
    return k(col3, ew3)


def _agg_body(hs_hbm, row_hbm, col_hbm, ew_hbm, parts_hbm,
              row_v, col_v, ew_v, rows_buf, accum, sem):
    c = lax.axis_index("c")
    s = lax.axis_index("s")
    wid = s * 2 + c
    zero = jnp.zeros((16,), jnp.float32)

    def zrow(e, carry):
        for t in range(_D // 16):
            rows_buf[e, pl.ds(t * 16, 16)] = zero
        return carry

    lax.fori_loop(0, _CH, zrow, 0)
    for k in range(_ROWS_PER_TILE // _CH):
        pltpu.sync_copy(rows_buf, accum.at[pl.ds(s * _ROWS_PER_TILE + k * _CH, _CH)])
    plsc.subcore_barrier()

    def blk(b, carry):
        pltpu.sync_copy(row_hbm.at[wid * _NBLK + b], row_v)
        pltpu.sync_copy(col_hbm.at[wid * _NBLK + b], col_v)
        pltpu.sync_copy(ew_hbm.at[wid * _NBLK + b], ew_v)

        def chunk(j, carry2):
            pltpu.async_copy(hs_hbm.at[row_v.at[j]], rows_buf, sem).wait()
            for g in range(_CH // 16):
                ewv = ew_v[j, pl.ds(g * 16, 16)]
                for l in range(16):
                    w = ewv[l]
                    e = g * 16 + l
                    for t in range(_D // 16):
                        sl = pl.ds(t * 16, 16)
                        rows_buf[e, sl] = rows_buf[e, sl] * w
            pltpu.sync_copy(rows_buf, accum.at[col_v.at[j]], add=True)
            return carry2

        lax.fori_loop(0, _BCH, chunk, 0)
        return carry

    lax.fori_loop(0, _NBLK, blk, 0)
    plsc.subcore_barrier()
    pltpu.sync_copy(accum.at[pl.ds(s * _ROWS_PER_TILE, _ROWS_PER_TILE)],
                    parts_hbm.at[c, pl.ds(s * _ROWS_PER_TILE, _ROWS_PER_TILE)])


def _sc_aggregate(hs, row3, col3, ew3):
    k = pl.kernel(
        _agg_body,
        out_type=jax.ShapeDtypeStruct((2, _NPAD, _D), jnp.float32),
        mesh=_sc_mesh(),
        scratch_types=[
            pltpu.VMEM((_BCH, _CH), jnp.int32),
            pltpu.VMEM((_BCH, _CH), jnp.int32),
            pltpu.VMEM((_BCH, _CH), jnp.float32),
            pltpu.VMEM((_CH, _D), jnp.float32),
            pltpu.VMEM_SHARED((_NPAD, _D), jnp.float32),
            pltpu.SemaphoreType.DMA,
        ],
    )
    return k(hs, row3, col3, ew3)


_BLK = 1000


def _tc1_body(x_ref, w_ref, degp_ref, hs_ref, dis_ref):
    h = jnp.dot(x_ref[...], w_ref[...], precision=jax.lax.Precision.HIGHEST,
                preferred_element_type=jnp.float32)
    deg = 1.0 + (degp_ref[0] + degp_ref[1])[:, 0:1]
    dis = lax.rsqrt(deg)
    hs_ref[...] = h * dis
    dis_ref[...] = dis


def _tc1(x, w1t, degp):
    grid = (_N // _BLK,)
    return pl.pallas_call(
        _tc1_body,
        grid=grid,
        in_specs=[
            pl.BlockSpec((_BLK, _D), lambda i: (i, 0)),
            pl.BlockSpec((_D, _D), lambda i: (0, 0)),
            pl.BlockSpec((2, _BLK, 16), lambda i: (0, i, 0)),
        ],
        out_specs=[
            pl.BlockSpec((_BLK, _D), lambda i: (i, 0)),
            pl.BlockSpec((_BLK, 1), lambda i: (i, 0)),
        ],
        out_shape=[
            jax.ShapeDtypeStruct((_N, _D), jnp.float32),
            jax.ShapeDtypeStruct((_N, 1), jnp.float32),
        ],
    )(x, w1t, degp)


def _tc2_body(p_ref, hs_ref, dis_ref, b_ref, w_ref, y_ref, hs2_ref):
    dis = dis_ref[...]
    y = jnp.tanh(dis * (p_ref[0] + p_ref[1] + hs_ref[...]) + b_ref[...])
    y_ref[...] = y
    h2 = jnp.dot(y, w_ref[...], precision=jax.lax.Precision.HIGHEST,
                 preferred_element_type=jnp.float32)
    hs2_ref[...] = h2 * dis


def _tc2(parts, hs1, dis, b1, w2t):
    grid = (_N // _BLK,)
    return pl.pallas_call(
        _tc2_body,
        grid=grid,
        in_specs=[
            pl.BlockSpec((2, _BLK, _D), lambda i: (0, i, 0)),
            pl.BlockSpec((_BLK, _D), lambda i: (i, 0)),
            pl.BlockSpec((_BLK, 1), lambda i: (i, 0)),
            pl.BlockSpec((1, _D), lambda i: (0, 0)),
            pl.BlockSpec((_D, _D), lambda i: (0, 0)),
        ],
        out_specs=[
            pl.BlockSpec((_BLK, _D), lambda i: (i, 0)),
            pl.BlockSpec((_BLK, _D), lambda i: (i, 0)),
        ],
        out_shape=[
            jax.ShapeDtypeStruct((_N, _D), jnp.float32),
            jax.ShapeDtypeStruct((_N, _D), jnp.float32),
        ],
    )(parts, hs1, dis, b1, w2t)


def _tc3_body(p_ref, hs_ref, dis_ref, b_ref, y_ref):
    y_ref[...] = jnp.tanh(
        dis_ref[...] * (p_ref[0] + p_ref[1] + hs_ref[...]) + b_ref[...])


def _tc3(parts, hs2, dis, b2):
    grid = (_N // _BLK,)
    return pl.pallas_call(
        _tc3_body,
        grid=grid,
        in_specs=[
            pl.BlockSpec((2, _BLK, _D), lambda i: (0, i, 0)),
            pl.BlockSpec((_BLK, _D), lambda i: (i, 0)),
            pl.BlockSpec((_BLK, 1), lambda i: (i, 0)),
            pl.BlockSpec((1, _D), lambda i: (0, 0)),
        ],
        out_specs=pl.BlockSpec((_BLK, _D), lambda i: (i, 0)),
        out_shape=jax.ShapeDtypeStruct((_N, _D), jnp.float32),
    )(parts, hs2, dis, b2)


def kernel(x, edge_index, edge_weight, W1, b1, W2, b2):
    row3 = edge_index[0].reshape(_NW * _NBLK, _BCH, _CH)
    col3 = edge_index[1].reshape(_NW * _NBLK, _BCH, _CH)
    ew3 = edge_weight.reshape(_NW * _NBLK, _BCH, _CH)

    degp = _sc_degree(col3, ew3)
    hs1, dis = _tc1(x, W1.T, degp)
    parts1 = _sc_aggregate(hs1, row3, col3, ew3)
    y1, hs2 = _tc2(parts1, hs1, dis, b1.reshape(1, _D), W2.T)
    parts2 = _sc_aggregate(hs2, row3, col3, ew3)
    y2 = _tc3(parts2, hs2, dis, b2.reshape(1, _D))
    return jnp.stack([x, y1, y2], axis=0)
